# baseline (speedup 1.0000x reference)
import functools
import math

import jax
import jax.numpy as jnp
from jax import lax
from jax.experimental import pallas as pl
from jax.experimental.pallas import tpu as pltpu
from jax.experimental.pallas import tpu_sc as plsc

D = 1024
B = 32
SEQ = 1024
OFFSET = 2
EPS = 1e-5

NW = 32
NSPLIT = 4
SROWS = SEQ // NSPLIT
CROWS = SROWS * B
ROWS_PER_W = CROWS // NW
CHUNK = 32
NCHUNK = ROWS_PER_W // CHUNK


def _sc_gather(tokens_t, weight):
    mesh = plsc.VectorSubcoreMesh(core_axis_name="c", subcore_axis_name="s")

    @functools.partial(
        pl.kernel,
        out_type=jax.ShapeDtypeStruct((CROWS, D), jnp.float32),
        mesh=mesh,
        scratch_types=[
            pltpu.VMEM((ROWS_PER_W,), jnp.int32),
            pltpu.VMEM((CHUNK, D), jnp.float32),
            pltpu.VMEM((CHUNK, D), jnp.float32),
            pltpu.SemaphoreType.DMA,
            pltpu.SemaphoreType.DMA,
        ],
    )
    def gather_kernel(tok_hbm, w_hbm, out_hbm, tok_v, buf_a, buf_b, sem_a, sem_b):
        wid = lax.axis_index("c") * 16 + lax.axis_index("s")
        base = wid * ROWS_PER_W
        pltpu.sync_copy(tok_hbm.at[pl.ds(base, ROWS_PER_W)], tok_v)
        bufs = (buf_a, buf_b)
        sems = (sem_a, sem_b)
        descs = [None, None]

        def start(c):
            p = c % 2
            idx = tok_v.at[pl.ds(c * CHUNK, CHUNK)]
            descs[p] = pltpu.async_copy(w_hbm.at[idx], bufs[p], sems[p])

        start(0)
        for c in range(NCHUNK):
            p = c % 2
            if c + 1 < NCHUNK:
                start(c + 1)
            descs[p].wait()
            pltpu.sync_copy(bufs[p], out_hbm.at[pl.ds(base + c * CHUNK, CHUNK)])

    return gather_kernel(tokens_t, weight)


SBLK = 32
NBLK = SROWS // SBLK


def _ln_body(emb_ref, pos_ref, g_ref, b_ref, out_ref):
    x = emb_ref[...].reshape(SBLK, B, D) * math.sqrt(D)
    x = x + pos_ref[...][:, None, :]
    mean = jnp.mean(x, axis=-1, keepdims=True)
    var = jnp.mean((x - mean) ** 2, axis=-1, keepdims=True)
    y = (x - mean) * lax.rsqrt(var + EPS) * g_ref[...] + b_ref[...]
    out_ref[...] = y.reshape(SBLK * B, D)


def _tc_ln_chunk(embed_k, pos_k, g2, b2, buf, k):
    def body(emb_ref, pos_ref, g_ref, b_ref, *rest):
        _ln_body(emb_ref, pos_ref, g_ref, b_ref, rest[-1])

    in_specs = [
        pl.BlockSpec((SBLK * B, D), lambda i: (i, 0)),
        pl.BlockSpec((SBLK, D), lambda i: (i, 0)),
        pl.BlockSpec((1, D), lambda i: (0, 0)),
        pl.BlockSpec((1, D), lambda i: (0, 0)),
    ]
    args = [embed_k, pos_k, g2, b2]
    aliases = {}
    if buf is not None:
        in_specs.append(pl.BlockSpec(memory_space=pl.ANY))
        args.append(buf)
        aliases = {4: 0}
    return pl.pallas_call(
        body,
        grid=(NBLK,),
        in_specs=in_specs,
        out_specs=pl.BlockSpec((SBLK * B, D), lambda i, k=k: (k * NBLK + i, 0)),
        out_shape=jax.ShapeDtypeStruct((SEQ * B, D), jnp.float32),
        input_output_aliases=aliases,
    )(*args)


def kernel(tokens, weight, pos_table, gamma, beta):
    tokens_t = tokens.T.reshape(-1)
    pos_sl = lax.slice_in_dim(pos_table, OFFSET, OFFSET + SEQ, axis=0)
    g2 = gamma.reshape(1, D)
    b2 = beta.reshape(1, D)
    embeds = [
        _sc_gather(lax.slice_in_dim(tokens_t, k * CROWS, (k + 1) * CROWS, axis=0), weight)
        for k in range(NSPLIT)
    ]
    buf = None
    for k in range(NSPLIT):
        pos_k = lax.slice_in_dim(pos_sl, k * SROWS, (k + 1) * SROWS, axis=0)
        buf = _tc_ln_chunk(embeds[k], pos_k, g2, b2, buf, k)
    return buf.reshape(SEQ, B, D)
